# baseline (speedup 1.0000x reference)
import functools

import jax
import jax.numpy as jnp
from jax import lax
from jax.experimental import pallas as pl
from jax.experimental.pallas import tpu as pltpu
from jax.experimental.pallas import tpu_sc as plsc

VOCAB = 18
FEAT = 64
BATCH = 16384
HIST = 50
NC, NS, L = 2, 16, 16
NW = NC * NS
NBT = BATCH // 128
BTW = NBT // NW
BW = BTW * 128
NFT = FEAT // 8
TSTRIDE = 80
TSZ = VOCAB * TSTRIDE
VPAD = 32
BTG = 4


def _unit(idx_v, tsw, buf, h):
    def btg_body(btg, c0):
        def j_body(j, c1):
            i_vec = idx_v[h, pl.ds(btg * 128 + j * L, L)]
            ibase = i_vec * TSTRIDE + (i_vec & 15)

            @plsc.parallel_loop(0, FEAT, unroll=8)
            def f_loop(f):
                v = plsc.load_gather(tsw, [ibase + f])
                buf[f >> 3, btg, f & 7, pl.ds(j * L, L)] = v

            return c1
        return lax.fori_loop(0, 8, j_body, c0)
    lax.fori_loop(0, BTW, btg_body, 0)


def _sc_body(w_hbm, p_hbm, op_hbm, idx_v, table_v, tsw, bufs, ssem):
    wid = lax.axis_index("s") * NC + lax.axis_index("c")
    bt0 = wid * BTW
    col0 = wid * BW

    pltpu.sync_copy(w_hbm, table_v)
    iota = lax.iota(jnp.int32, L)
    for r in range(VOCAB):
        base = r * TSTRIDE + (r & 15)
        for k in range(FEAT // L):
            plsc.store_scatter(
                tsw, [iota + (base + k * L)], table_v[r, pl.ds(k * L, L)])

    def scatter(out_hbm, h, slot):
        pltpu.async_copy(
            bufs.at[slot], out_hbm.at[h, :, pl.ds(bt0, BTW)], ssem.at[slot])

    def scatter_wait(out_hbm, slot):
        pltpu.make_async_copy(
            bufs.at[slot], out_hbm.at[0, :, pl.ds(bt0, BTW)],
            ssem.at[slot]).wait()

    pltpu.sync_copy(p_hbm.at[:, pl.ds(col0, BW)], idx_v)
    for slot in range(2):
        _unit(idx_v, tsw, bufs.at[slot], slot)
        scatter(op_hbm, slot, slot)

    def pair(hh, carry):
        for slot in range(2):
            h = hh * 2 + slot
            scatter_wait(op_hbm, slot)
            _unit(idx_v, tsw, bufs.at[slot], h)
            scatter(op_hbm, h, slot)
        return carry

    lax.fori_loop(1, HIST // 2, pair, 0)
    for slot in range(2):
        scatter_wait(op_hbm, slot)


def _tc_body(wt_ref, idx_ref, o_ref):
    idx = idx_ref[0]
    rows = lax.broadcasted_iota(jnp.int32, (VPAD, BTG * 128), 0)
    onehot = (jnp.broadcast_to(idx, (VPAD, BTG * 128)) == rows)
    res = jnp.dot(wt_ref[...], onehot.astype(jnp.bfloat16),
                  preferred_element_type=jnp.float32)
    for btg in range(BTG):
        o_ref[0, :, btg] = res[:, btg * 128:(btg + 1) * 128].reshape(
            NFT, 8, 128)


@jax.jit
def _lookup(patch_t, context_t, weight):
    mesh = plsc.VectorSubcoreMesh(core_axis_name="c", subcore_axis_name="s")
    out5 = jax.ShapeDtypeStruct((HIST, NFT, NBT, 8, 128), jnp.float32)
    sc = pl.kernel(
        _sc_body,
        out_type=out5,
        mesh=mesh,
        scratch_types=[
            pltpu.VMEM((HIST, BW), jnp.int32),
            pltpu.VMEM((VOCAB, FEAT), jnp.float32),
            pltpu.VMEM((TSZ,), jnp.float32),
            pltpu.VMEM((2, NFT, BTW, 8, 128), jnp.float32),
            pltpu.SemaphoreType.DMA((2,)),
        ],
        compiler_params=pltpu.CompilerParams(
            use_tc_tiling_on_sc=False, needs_layout_passes=False),
    )
    wt_pad = (jnp.zeros((FEAT, VPAD), jnp.float32).at[:, :VOCAB]
              .set(weight.T).astype(jnp.bfloat16))
    idx3 = context_t.reshape(HIST * (NBT // BTG), 1, BTG * 128)
    out5_c = pl.pallas_call(
        _tc_body,
        out_shape=out5,
        grid=(HIST, NBT // BTG),
        in_specs=[
            pl.BlockSpec((FEAT, VPAD), lambda h, b: (0, 0)),
            pl.BlockSpec((1, 1, BTG * 128),
                         lambda h, b: (h * (NBT // BTG) + b, 0, 0)),
        ],
        out_specs=pl.BlockSpec((1, NFT, BTG, 8, 128),
                               lambda h, b: (h, 0, b, 0, 0)),
    )(wt_pad, idx3)
    out5_p = sc(weight, patch_t)
    return out5_p, out5_c


def kernel(patch, context, weight):
    out5_p, out5_c = _lookup(patch.T, context.T, weight)

    def relabel(o5):
        return o5.transpose((2, 4, 0, 1, 3)).reshape(BATCH, HIST, FEAT)

    return (relabel(out5_p), relabel(out5_c))


# speedup vs baseline: 5.3948x; 5.3948x over previous
import functools

import jax
import jax.numpy as jnp
from jax import lax
from jax.experimental import pallas as pl
from jax.experimental.pallas import tpu as pltpu
from jax.experimental.pallas import tpu_sc as plsc

VOCAB = 18
FEAT = 64
BATCH = 16384
HIST = 50
NC, NS, L = 2, 16, 16
NW = NC * NS
NBT = BATCH // 128
BTW = NBT // NW
BW = BTW * 128
NFT = FEAT // 8
TSTRIDE = 80
TSZ = VOCAB * TSTRIDE
VPAD = 32
BTG = 128


def _unit(idx_v, tsw, buf, h):
    def btg_body(btg, c0):
        def j_body(j, c1):
            i_vec = idx_v[h, pl.ds(btg * 128 + j * L, L)]
            ibase = i_vec * TSTRIDE + (i_vec & 15)

            @plsc.parallel_loop(0, FEAT, unroll=8)
            def f_loop(f):
                v = plsc.load_gather(tsw, [ibase + f])
                buf[f >> 3, btg, f & 7, pl.ds(j * L, L)] = v

            return c1
        return lax.fori_loop(0, 8, j_body, c0)
    lax.fori_loop(0, BTW, btg_body, 0)


def _sc_body(w_hbm, p_hbm, op_hbm, idx_v, table_v, tsw, bufs, ssem):
    wid = lax.axis_index("s") * NC + lax.axis_index("c")
    bt0 = wid * BTW
    col0 = wid * BW

    pltpu.sync_copy(w_hbm, table_v)
    iota = lax.iota(jnp.int32, L)
    for r in range(VOCAB):
        base = r * TSTRIDE + (r & 15)
        for k in range(FEAT // L):
            plsc.store_scatter(
                tsw, [iota + (base + k * L)], table_v[r, pl.ds(k * L, L)])

    def scatter(out_hbm, h, slot):
        pltpu.async_copy(
            bufs.at[slot], out_hbm.at[h, :, pl.ds(bt0, BTW)], ssem.at[slot])

    def scatter_wait(out_hbm, slot):
        pltpu.make_async_copy(
            bufs.at[slot], out_hbm.at[0, :, pl.ds(bt0, BTW)],
            ssem.at[slot]).wait()

    pltpu.sync_copy(p_hbm.at[:, pl.ds(col0, BW)], idx_v)
    for slot in range(2):
        _unit(idx_v, tsw, bufs.at[slot], slot)
        scatter(op_hbm, slot, slot)

    def pair(hh, carry):
        for slot in range(2):
            h = hh * 2 + slot
            scatter_wait(op_hbm, slot)
            _unit(idx_v, tsw, bufs.at[slot], h)
            scatter(op_hbm, h, slot)
        return carry

    lax.fori_loop(1, HIST // 2, pair, 0)
    for slot in range(2):
        scatter_wait(op_hbm, slot)


def _tc_body(wt_ref, idx_ref, o_ref):
    idx = idx_ref[0]
    rows = lax.broadcasted_iota(jnp.int32, (VPAD, BTG * 128), 0)
    onehot = (jnp.broadcast_to(idx, (VPAD, BTG * 128)) == rows)
    res = jnp.dot(wt_ref[...], onehot.astype(jnp.bfloat16),
                  preferred_element_type=jnp.float32)
    for btg in range(BTG):
        o_ref[0, :, btg] = res[:, btg * 128:(btg + 1) * 128].reshape(
            NFT, 8, 128)


@jax.jit
def _lookup(patch_t, context_t, weight):
    mesh = plsc.VectorSubcoreMesh(core_axis_name="c", subcore_axis_name="s")
    out5 = jax.ShapeDtypeStruct((HIST, NFT, NBT, 8, 128), jnp.float32)
    sc = pl.kernel(
        _sc_body,
        out_type=out5,
        mesh=mesh,
        scratch_types=[
            pltpu.VMEM((HIST, BW), jnp.int32),
            pltpu.VMEM((VOCAB, FEAT), jnp.float32),
            pltpu.VMEM((TSZ,), jnp.float32),
            pltpu.VMEM((2, NFT, BTW, 8, 128), jnp.float32),
            pltpu.SemaphoreType.DMA((2,)),
        ],
        compiler_params=pltpu.CompilerParams(
            use_tc_tiling_on_sc=False, needs_layout_passes=False),
    )
    wt_pad = (jnp.zeros((FEAT, VPAD), jnp.float32).at[:, :VOCAB]
              .set(weight.T).astype(jnp.bfloat16))
    idx3 = context_t.reshape(HIST * (NBT // BTG), 1, BTG * 128)
    out5_c = pl.pallas_call(
        _tc_body,
        out_shape=out5,
        grid=(HIST, NBT // BTG),
        in_specs=[
            pl.BlockSpec((FEAT, VPAD), lambda h, b: (0, 0)),
            pl.BlockSpec((1, 1, BTG * 128),
                         lambda h, b: (h * (NBT // BTG) + b, 0, 0)),
        ],
        out_specs=pl.BlockSpec((1, NFT, BTG, 8, 128),
                               lambda h, b: (h, 0, b, 0, 0)),
    )(wt_pad, idx3)
    out5_p = sc(weight, patch_t)
    return out5_p, out5_c


def kernel(patch, context, weight):
    out5_p, out5_c = _lookup(patch.T, context.T, weight)

    def relabel(o5):
        return o5.transpose((2, 4, 0, 1, 3)).reshape(BATCH, HIST, FEAT)

    return (relabel(out5_p), relabel(out5_c))
